# Initial kernel scaffold; baseline (speedup 1.0000x reference)
import jax
import jax.numpy as jnp
from jax.experimental import pallas as pl


def kernel(x, edge_index, W1_l, b1_l, W1_r, gamma1, beta1, W2_l, b2_l, W2_r):
    raise NotImplementedError("write your pallas kernel here")


# speedup vs baseline: 5.4691x; 5.4691x over previous
import dataclasses
import functools

import jax
import jax.numpy as jnp
from jax import lax
from jax.experimental import pallas as pl
from jax.experimental.pallas import tpu as pltpu
from jax.experimental.pallas import tpu_sc as plsc

N = 10000
E = 320000
D = 128
BN_SCALE = float(1.0 / (1.0 + 1e-5) ** 0.5)

NC = 2
NS = 16
NW = NC * NS
CHUNK = 128
NCHUNK = 79
EPAD = NW * NCHUNK * CHUNK
EPW = EPAD // NW
NPAD = 10240
ROWS_PER_TILE = NPAD // NS
ZROWS = 64
CNT_W = 16

BLK = 2048


def _mm2_body(x_ref, wl_ref, wr_ref, b_ref, y_ref, z_ref):
    x = x_ref[...]
    dn = (((1,), (1,)), ((), ()))
    y_ref[...] = lax.dot_general(x, wl_ref[...], dn,
                                 preferred_element_type=jnp.float32)
    z_ref[...] = lax.dot_general(x, wr_ref[...], dn,
                                 preferred_element_type=jnp.float32) + b_ref[...]


def _mm2(x, wl, wr, b):
    return pl.pallas_call(
        _mm2_body,
        grid=(pl.cdiv(N, BLK),),
        in_specs=[
            pl.BlockSpec((BLK, D), lambda i: (i, 0)),
            pl.BlockSpec((D, D), lambda i: (0, 0)),
            pl.BlockSpec((D, D), lambda i: (0, 0)),
            pl.BlockSpec((1, D), lambda i: (0, 0)),
        ],
        out_specs=[
            pl.BlockSpec((BLK, D), lambda i: (i, 0)),
            pl.BlockSpec((BLK, D), lambda i: (i, 0)),
        ],
        out_shape=[jax.ShapeDtypeStruct((N, D), jnp.float32)] * 2,
    )(x, wl, wr, b)


def _epilogue1_body(sp_ref, cp_ref, z1_ref, g_ref, be_ref, wl_ref, wr_ref,
                    b2_ref, y2_ref, z2_ref):
    s = sp_ref[0] + sp_ref[1]
    dncnt = (((0,), (0,)), ((), ()))
    cnt = lax.dot_general(cp_ref[...], jnp.ones((NW, 1), jnp.float32), dncnt,
                          preferred_element_type=jnp.float32)
    mean = s / jnp.clip(cnt, 1.0, None)
    h = mean + z1_ref[...]
    h = (h * BN_SCALE) * g_ref[...] + be_ref[...]
    h = jnp.maximum(h, 0.0)
    dn = (((1,), (1,)), ((), ()))
    y2_ref[...] = lax.dot_general(h, wl_ref[...], dn,
                                  preferred_element_type=jnp.float32)
    z2_ref[...] = lax.dot_general(h, wr_ref[...], dn,
                                  preferred_element_type=jnp.float32) + b2_ref[...]


def _epilogue1(s_parts, cnt_parts, z1, gamma, beta, w2l, w2r, b2):
    return pl.pallas_call(
        _epilogue1_body,
        grid=(pl.cdiv(N, BLK),),
        in_specs=[
            pl.BlockSpec((NC, BLK, D), lambda i: (0, i, 0)),
            pl.BlockSpec((NW, BLK), lambda i: (0, i)),
            pl.BlockSpec((BLK, D), lambda i: (i, 0)),
            pl.BlockSpec((1, D), lambda i: (0, 0)),
            pl.BlockSpec((1, D), lambda i: (0, 0)),
            pl.BlockSpec((D, D), lambda i: (0, 0)),
            pl.BlockSpec((D, D), lambda i: (0, 0)),
            pl.BlockSpec((1, D), lambda i: (0, 0)),
        ],
        out_specs=[
            pl.BlockSpec((BLK, D), lambda i: (i, 0)),
            pl.BlockSpec((BLK, D), lambda i: (i, 0)),
        ],
        out_shape=[jax.ShapeDtypeStruct((N, D), jnp.float32)] * 2,
    )(s_parts, cnt_parts, z1, gamma, beta, w2l, w2r, b2)


def _epilogue2_body(sp_ref, cp_ref, z2_ref, o_ref):
    s = sp_ref[0] + sp_ref[1]
    dncnt = (((0,), (0,)), ((), ()))
    cnt = lax.dot_general(cp_ref[...], jnp.ones((NW, 1), jnp.float32), dncnt,
                          preferred_element_type=jnp.float32)
    o_ref[...] = s / jnp.clip(cnt, 1.0, None) + z2_ref[...]


def _epilogue2(s_parts, cnt_parts, z2):
    return pl.pallas_call(
        _epilogue2_body,
        grid=(pl.cdiv(N, BLK),),
        in_specs=[
            pl.BlockSpec((NC, BLK, D), lambda i: (0, i, 0)),
            pl.BlockSpec((NW, BLK), lambda i: (0, i)),
            pl.BlockSpec((BLK, D), lambda i: (i, 0)),
        ],
        out_specs=pl.BlockSpec((BLK, D), lambda i: (i, 0)),
        out_shape=jax.ShapeDtypeStruct((N, D), jnp.float32),
    )(s_parts, cnt_parts, z2)


def _zero_vmem(buf, rows, width):
    zv = jnp.zeros((16,), jnp.float32)

    @pl.loop(0, rows)
    def _(i):
        @pl.loop(0, width // 16)
        def _(j):
            buf[i, pl.ds(j * 16, 16)] = zv


def _make_segsum():
    mesh = plsc.VectorSubcoreMesh(core_axis_name="c", subcore_axis_name="s")
    scratch = [
        pltpu.VMEM((NCHUNK, CHUNK), jnp.int32),
        pltpu.VMEM((NCHUNK, CHUNK), jnp.int32),
        pltpu.VMEM((CHUNK, D), jnp.float32),
        pltpu.VMEM((ZROWS, D), jnp.float32),
        pltpu.VMEM_SHARED((NPAD, D), jnp.float32),
    ]

    def body(y_hbm, src_hbm, dst_hbm, out_hbm, src_v, dst_v, rows_v, zb_v, acc):
        c = lax.axis_index("c")
        s = lax.axis_index("s")
        wid = c * NS + s
        r0 = s * ROWS_PER_TILE

        _zero_vmem(zb_v, ZROWS, D)

        @pl.loop(0, ROWS_PER_TILE // ZROWS)
        def _(i):
            pltpu.sync_copy(zb_v, acc.at[pl.ds(r0 + i * ZROWS, ZROWS)])

        pltpu.sync_copy(src_hbm.at[wid], src_v)
        pltpu.sync_copy(dst_hbm.at[wid], dst_v)

        plsc.subcore_barrier()

        @pl.loop(0, NCHUNK)
        def _(k):
            pltpu.sync_copy(y_hbm.at[src_v.at[k]], rows_v)
            pltpu.sync_copy(rows_v, acc.at[dst_v.at[k]], add=True)

        plsc.subcore_barrier()

        pltpu.sync_copy(acc.at[pl.ds(r0, ROWS_PER_TILE)],
                        out_hbm.at[c, pl.ds(r0, ROWS_PER_TILE)])

    return pl.kernel(body,
                     out_type=jax.ShapeDtypeStruct((NC, NPAD, D), jnp.float32),
                     mesh=mesh, scratch_types=scratch)


def _make_counts():
    mesh = plsc.VectorSubcoreMesh(core_axis_name="c", subcore_axis_name="s")
    scratch = [
        pltpu.VMEM((NCHUNK, CHUNK), jnp.int32),
        pltpu.VMEM((NPAD,), jnp.float32),
    ]

    def body(dst_hbm, cnt_hbm, dst_v, hist):
        c = lax.axis_index("c")
        s = lax.axis_index("s")
        wid = c * NS + s

        zv = jnp.zeros((16,), jnp.float32)

        @pl.loop(0, NPAD // 16)
        def _(i):
            hist[pl.ds(i * 16, 16)] = zv

        pltpu.sync_copy(dst_hbm.at[wid], dst_v)

        ov = jnp.ones((16,), jnp.float32)

        @pl.loop(0, NCHUNK)
        def _(k):
            @pl.loop(0, CHUNK // 16)
            def _(j):
                iv = dst_v[k, pl.ds(j * 16, 16)]
                plsc.addupdate_scatter(hist, [iv], ov)

        pltpu.sync_copy(hist, cnt_hbm.at[wid])

    cp = pltpu.CompilerParams()
    if "needs_layout_passes" in pltpu.CompilerParams.__dataclass_fields__:
        cp = dataclasses.replace(cp, needs_layout_passes=False)
    return pl.kernel(body,
                     out_type=jax.ShapeDtypeStruct((NW, NPAD), jnp.float32),
                     mesh=mesh, scratch_types=scratch, compiler_params=cp)


_segsum = _make_segsum()
_counts = _make_counts()


@jax.jit
def kernel(x, edge_index, W1_l, b1_l, W1_r, gamma1, beta1, W2_l, b2_l, W2_r):
    npad_e = EPAD - E
    src = jnp.concatenate(
        [edge_index[0].astype(jnp.int32), jnp.zeros((npad_e,), jnp.int32)]
    ).reshape(NW, NCHUNK, CHUNK)
    dst = jnp.concatenate(
        [edge_index[1].astype(jnp.int32),
         jnp.full((npad_e,), N, jnp.int32)]
    ).reshape(NW, NCHUNK, CHUNK)
    b1 = b1_l.reshape(1, D)
    b2 = b2_l.reshape(1, D)
    g1 = gamma1.reshape(1, D)
    be1 = beta1.reshape(1, D)

    cntp = _counts(dst)
    y1, z1 = _mm2(x, W1_l, W1_r, b1)
    s1p = _segsum(y1, src, dst)
    y2, z2 = _epilogue1(s1p, cntp, z1, g1, be1, W2_l, W2_r, b2)
    s2p = _segsum(y2, src, dst)
    return _epilogue2(s2p, cntp, z2)
